# baseline (speedup 1.0000x reference)
import functools

import jax
import jax.numpy as jnp
from jax import lax
from jax.experimental import pallas as pl
from jax.experimental.pallas import tpu as pltpu
from jax.experimental.pallas import tpu_sc as plsc

NC = 2
NS = 16
NW = NC * NS
CHUNK = 128
K = 13


def _make_sc_gather(n_total, dim):
    per_w = n_total // NW
    n_chunks = per_w // CHUNK
    n_groups = n_chunks // K
    rows_per_group = K * CHUNK
    mesh = plsc.VectorSubcoreMesh(core_axis_name="c", subcore_axis_name="s")

    @functools.partial(
        pl.kernel,
        out_type=jax.ShapeDtypeStruct((n_total, dim), jnp.float32),
        mesh=mesh,
        scratch_types=[
            pltpu.VMEM((n_chunks, CHUNK), jnp.int32),
            pltpu.VMEM((rows_per_group, dim), jnp.float32),
            pltpu.VMEM((rows_per_group, dim), jnp.float32),
            pltpu.SemaphoreType.DMA,
            pltpu.SemaphoreType.DMA,
            pltpu.SemaphoreType.DMA,
            pltpu.SemaphoreType.DMA,
        ],
        compiler_params=pltpu.CompilerParams(use_tc_tiling_on_sc=False),
    )
    def body(idx_hbm, tab_hbm, out_hbm, idx_v, rows0, rows1, g0, g1, o0, o1):
        wid = lax.axis_index("s") * NC + lax.axis_index("c")
        base = wid * per_w
        pltpu.sync_copy(idx_hbm.at[wid], idx_v)

        rows = (rows0, rows1)
        gsem = (g0, g1)
        osem = (o0, o1)

        def fire_gather(g, b):
            for j in range(K):
                pltpu.async_copy(
                    tab_hbm.at[idx_v.at[g * K + j]],
                    rows[b].at[pl.ds(j * CHUNK, CHUNK)],
                    gsem[b],
                )

        def drain_gather(b):
            pltpu.make_async_copy(tab_hbm.at[pl.ds(0, rows_per_group)],
                                  rows[b], gsem[b]).wait()

        def fire_out(g, b):
            pltpu.async_copy(
                rows[b],
                out_hbm.at[pl.ds(base + g * rows_per_group, rows_per_group)],
                osem[b],
            )

        def wait_out(b):
            pltpu.make_async_copy(rows[b],
                                  out_hbm.at[pl.ds(base, rows_per_group)],
                                  osem[b]).wait()

        fire_gather(0, 0)

        def step(g, carry):
            b = g % 2

            def one(bb):
                drain_gather(bb)
                fire_out(g, bb)

                @pl.when(g + 1 < n_groups)
                def _():
                    @pl.when(g >= 1)
                    def _():
                        wait_out(1 - bb)
                    fire_gather(g + 1, 1 - bb)

            @pl.when(b == 0)
            def _():
                one(0)

            @pl.when(b == 1)
            def _():
                one(1)

            return carry

        lax.fori_loop(0, n_groups, step, 0)
        wait_out(0)
        wait_out(1)

    return body


def kernel(x, E):
    b, f = x.shape
    v, d = E.shape
    n_total = b * f
    xf = x.astype(jnp.int32).reshape(NW, n_total // (NW * CHUNK), CHUNK)
    out = _make_sc_gather(n_total, d)(xf, E)
    return out.reshape(b, f, d)
